# Initial kernel scaffold; baseline (speedup 1.0000x reference)
import jax
import jax.numpy as jnp
from jax.experimental import pallas as pl


def kernel(x, adj, labels, W1, b1, W2, b2, W3, b3):
    raise NotImplementedError("write your pallas kernel here")


# speedup vs baseline: 1.0814x; 1.0814x over previous
import functools

import jax
import jax.numpy as jnp
from jax.experimental import pallas as pl


def _pass1_body(adj_ref, xb_ref, w1_ref, b1_ref, w2_ref, g2_ref, adjb_ref):
    ab = adj_ref[...].astype(jnp.bfloat16)
    adjb_ref[...] = ab
    t = jnp.dot(ab, xb_ref[...], preferred_element_type=jnp.float32)
    h = jnp.maximum(
        jnp.dot(t, w1_ref[...], preferred_element_type=jnp.float32) + b1_ref[...],
        0.0,
    )
    g2_ref[...] = jnp.dot(h, w2_ref[...], preferred_element_type=jnp.float32).astype(
        jnp.bfloat16
    )


def _pass2_body(adjb_ref, g2_ref, b2_ref, w3_ref, g3_ref):
    t = jnp.dot(adjb_ref[...], g2_ref[...], preferred_element_type=jnp.float32)
    h = jnp.maximum(t + b2_ref[...], 0.0)
    g3_ref[...] = jnp.dot(h, w3_ref[...], preferred_element_type=jnp.float32).astype(
        jnp.bfloat16
    )


def _pass3_body(adjb_ref, g3_ref, b3_ref, out_ref):
    out_ref[...] = (
        jnp.dot(adjb_ref[...], g3_ref[...], preferred_element_type=jnp.float32)
        + b3_ref[...]
    )


def _block_rows(n: int) -> int:
    for bm in (400, 80, 16, 8):
        if n % bm == 0:
            return bm
    return n


@jax.jit
def kernel(x, adj, labels, W1, b1, W2, b2, W3, b3):
    del labels
    n, nfeat = x.shape
    nhid = W1.shape[1]
    ncls = W3.shape[1]
    bm = _block_rows(n)
    grid = (n // bm,)

    xb = x.astype(jnp.bfloat16)
    b1r = b1.reshape(1, nhid)
    b2r = b2.reshape(1, nhid)
    b3r = b3.reshape(1, ncls)

    full = lambda shape: pl.BlockSpec(shape, lambda i: (0, 0))
    rows = lambda cols: pl.BlockSpec((bm, cols), lambda i: (i, 0))

    g2, adjb = pl.pallas_call(
        _pass1_body,
        grid=grid,
        in_specs=[
            rows(n),
            full((n, nfeat)),
            full((nfeat, nhid)),
            full((1, nhid)),
            full((nhid, nhid)),
        ],
        out_specs=[rows(nhid), rows(n)],
        out_shape=[
            jax.ShapeDtypeStruct((n, nhid), jnp.bfloat16),
            jax.ShapeDtypeStruct((n, n), jnp.bfloat16),
        ],
    )(adj, xb, W1, b1r, W2)

    g3 = pl.pallas_call(
        _pass2_body,
        grid=grid,
        in_specs=[rows(n), full((n, nhid)), full((1, nhid)), full((nhid, ncls))],
        out_specs=rows(ncls),
        out_shape=jax.ShapeDtypeStruct((n, ncls), jnp.bfloat16),
    )(adjb, g2, b2r, W3)

    out = pl.pallas_call(
        _pass3_body,
        grid=grid,
        in_specs=[rows(n), full((n, ncls)), full((1, ncls))],
        out_specs=rows(ncls),
        out_shape=jax.ShapeDtypeStruct((n, ncls), jnp.float32),
    )(adjb, g3, b3r)
    return out
